# baseline (speedup 1.0000x reference)
import functools

import jax
import jax.numpy as jnp
from jax import lax
from jax.experimental import pallas as pl
from jax.experimental.pallas import tpu as pltpu
from jax.experimental.pallas import tpu_sc as plsc


def kernel(ctx, prefix, suffix):
    n_ctx, d = ctx.shape
    n_cls = prefix.shape[0]
    n_suf = suffix.shape[1]
    seq = 1 + n_ctx + n_suf
    row = seq * d
    ctx_sz = n_ctx * d
    suf_sz = n_suf * d

    info = plsc.get_sparse_core_info()
    ns = info.num_subcores
    nw = info.num_cores * ns
    cpw = (n_cls + nw - 1) // nw

    mesh = plsc.VectorSubcoreMesh(core_axis_name="c", subcore_axis_name="s")

    @functools.partial(
        pl.kernel,
        out_type=jax.ShapeDtypeStruct((n_cls * row,), jnp.float32),
        mesh=mesh,
        scratch_types=[
            pltpu.MemorySpace.VMEM_SHARED((ns, 2, row), jnp.float32),
            pltpu.SemaphoreType.DMA,
            pltpu.SemaphoreType.DMA,
            pltpu.SemaphoreType.DMA,
            pltpu.SemaphoreType.DMA,
        ],
    )
    def body(ctx_hbm, prefix_hbm, suffix_hbm, out_hbm, shared, si0, si1, so0, so1):
        sin = (si0, si1)
        sout = (so0, so1)
        sid = lax.axis_index("s")
        wid = sid * info.num_cores + lax.axis_index("c")
        base = wid * cpw
        bufs = (shared.at[sid, 0], shared.at[sid, 1])

        pltpu.sync_copy(ctx_hbm, bufs[0].at[pl.ds(d, ctx_sz)])
        pltpu.sync_copy(ctx_hbm, bufs[1].at[pl.ds(d, ctx_sz)])

        def cls(j):
            return jnp.minimum(base + j, n_cls - 1)

        in_descs = [None] * cpw
        out_descs = [None] * cpw

        n_in = 5
        suf_chunk = suf_sz // n_in
        n_out = 11
        out_chunk = row // n_out

        def fire_in(j):
            p = j % 2
            i = cls(j)
            descs = [
                pltpu.async_copy(
                    prefix_hbm.at[pl.ds(i * d, d)], bufs[p].at[pl.ds(0, d)], sin[p]
                )
            ]
            for c in range(n_in):
                descs.append(
                    pltpu.async_copy(
                        suffix_hbm.at[pl.ds(i * suf_sz + c * suf_chunk, suf_chunk)],
                        bufs[p].at[pl.ds(d + ctx_sz + c * suf_chunk, suf_chunk)],
                        sin[p],
                    )
                )
            in_descs[j] = descs

        def fire_out(j):
            p = j % 2
            o = cls(j) * row
            out_descs[j] = [
                pltpu.async_copy(
                    bufs[p].at[pl.ds(c * out_chunk, out_chunk)],
                    out_hbm.at[pl.ds(o + c * out_chunk, out_chunk)],
                    sout[p],
                )
                for c in range(n_out)
            ]

        fire_in(0)
        for j in range(cpw):
            if j >= 1:
                for dsc in out_descs[j - 1]:
                    dsc.wait()
            if j + 1 < cpw:
                fire_in(j + 1)
            for dsc in in_descs[j]:
                dsc.wait()
            fire_out(j)
        for dsc in out_descs[cpw - 1]:
            dsc.wait()

    flat = body(ctx.reshape(-1), prefix.reshape(-1), suffix.reshape(-1))
    return flat.reshape(n_cls, seq, d)


# speedup vs baseline: 1.0016x; 1.0016x over previous
import functools

import jax
import jax.numpy as jnp
from jax import lax
from jax.experimental import pallas as pl
from jax.experimental.pallas import tpu as pltpu
from jax.experimental.pallas import tpu_sc as plsc


def kernel(ctx, prefix, suffix):
    n_ctx, d = ctx.shape
    n_cls = prefix.shape[0]
    n_suf = suffix.shape[1]
    seq = 1 + n_ctx + n_suf

    info = plsc.get_sparse_core_info()
    ns = info.num_subcores
    nw = info.num_cores * ns
    cpw = (n_cls + nw - 1) // nw

    mesh = plsc.VectorSubcoreMesh(core_axis_name="c", subcore_axis_name="s")

    @functools.partial(
        pl.kernel,
        out_type=jax.ShapeDtypeStruct((n_cls, seq, d), jnp.float32),
        mesh=mesh,
        compiler_params=pltpu.CompilerParams(use_tc_tiling_on_sc=False),
        scratch_types=[
            pltpu.MemorySpace.VMEM_SHARED((ns, 2, seq, d), jnp.float32),
            pltpu.SemaphoreType.DMA,
            pltpu.SemaphoreType.DMA,
            pltpu.SemaphoreType.DMA,
            pltpu.SemaphoreType.DMA,
        ],
    )
    def body(ctx_hbm, prefix_hbm, suffix_hbm, out_hbm, shared, si0, si1, so0, so1):
        sin = (si0, si1)
        sout = (so0, so1)
        sid = lax.axis_index("s")
        wid = sid * info.num_cores + lax.axis_index("c")
        base = wid * cpw
        bufs = (shared.at[sid, 0], shared.at[sid, 1])

        pltpu.sync_copy(ctx_hbm, bufs[0].at[pl.ds(1, n_ctx)])
        pltpu.sync_copy(ctx_hbm, bufs[1].at[pl.ds(1, n_ctx)])

        def cls(j):
            return jnp.minimum(base + j, n_cls - 1)

        in_descs = [None] * cpw
        out_descs = [None] * cpw

        def fire_in(j):
            p = j % 2
            i = cls(j)
            d1 = pltpu.async_copy(
                prefix_hbm.at[i], bufs[p].at[pl.ds(0, 1)], sin[p]
            )
            d2 = pltpu.async_copy(
                suffix_hbm.at[i], bufs[p].at[pl.ds(1 + n_ctx, n_suf)], sin[p]
            )
            in_descs[j] = (d1, d2)

        def fire_out(j):
            p = j % 2
            out_descs[j] = pltpu.async_copy(bufs[p], out_hbm.at[cls(j)], sout[p])

        fire_in(0)
        for j in range(cpw):
            if j >= 1:
                out_descs[j - 1].wait()
            if j + 1 < cpw:
                fire_in(j + 1)
            in_descs[j][0].wait()
            in_descs[j][1].wait()
            fire_out(j)
        out_descs[cpw - 1].wait()

    return body(ctx, prefix, suffix)


# speedup vs baseline: 5.0765x; 5.0682x over previous
import functools

import jax
import jax.numpy as jnp
from jax import lax
from jax.experimental import pallas as pl
from jax.experimental.pallas import tpu as pltpu
from jax.experimental.pallas import tpu_sc as plsc

_W = 128
_L = 16


def kernel(ctx, prefix, suffix):
    n_ctx, d = ctx.shape
    n_cls = prefix.shape[0]
    n_suf = suffix.shape[1]
    seq = 1 + n_ctx + n_suf
    nc = d // _W

    info = plsc.get_sparse_core_info()
    nw = info.num_cores * info.num_subcores
    cpw = (n_cls + nw - 1) // nw

    mesh = plsc.VectorSubcoreMesh(core_axis_name="c", subcore_axis_name="s")

    @functools.partial(
        pl.kernel,
        out_type=jax.ShapeDtypeStruct((n_cls, seq, d), jnp.float32),
        mesh=mesh,
        scratch_types=[
            pltpu.VMEM((nc, seq, _W), jnp.float32),
            pltpu.VMEM((nc, n_suf, _W), jnp.float32),
            pltpu.VMEM((1, d), jnp.float32),
            pltpu.SemaphoreType.DMA,
            pltpu.SemaphoreType.DMA,
            pltpu.SemaphoreType.DMA,
        ],
    )
    def body(ctx_hbm, prefix_hbm, suffix_hbm, out_hbm, vbuf, vsuf, vpre,
             sem_in, sem_out, sem_pre):
        wid = lax.axis_index("s") * info.num_cores + lax.axis_index("c")
        base = wid * cpw

        def cls(j):
            return jnp.minimum(base + j, n_cls - 1)

        for c in range(nc):
            pltpu.sync_copy(
                ctx_hbm.at[:, pl.ds(c * _W, _W)], vsuf.at[c, pl.ds(0, n_ctx)]
            )

        def copy_row(dst, dst_row, src, src_row, r):
            for k in range(_W // _L):
                s = pl.ds(k * _L, _L)
                dst[dst_row + r, s] = src[src_row + r, s]

        def ctx_body(r, _):
            for c in range(nc):
                copy_row(vbuf.at[c], 1, vsuf.at[c], 0, r)
            return ()

        lax.fori_loop(0, n_ctx, ctx_body, (), unroll=False)

        def fire_in(j):
            i = cls(j)
            for c in range(nc):
                pltpu.async_copy(
                    suffix_hbm.at[i, :, pl.ds(c * _W, _W)],
                    vsuf.at[c, pl.ds(0, n_suf)],
                    sem_in,
                )

        def fire_pre(j):
            pltpu.async_copy(prefix_hbm.at[cls(j)], vpre, sem_pre)

        def wait_in():
            for c in range(nc):
                pltpu.make_async_copy(
                    suffix_hbm.at[0, :, pl.ds(0, _W)],
                    vsuf.at[c, pl.ds(0, n_suf)],
                    sem_in,
                ).wait()

        def wait_pre():
            pltpu.make_async_copy(prefix_hbm.at[0], vpre, sem_pre).wait()

        def fire_out(j):
            i = cls(j)
            for c in range(nc):
                pltpu.async_copy(
                    vbuf.at[c], out_hbm.at[i, :, pl.ds(c * _W, _W)], sem_out
                )

        def wait_out():
            for c in range(nc):
                pltpu.make_async_copy(
                    vbuf.at[c], out_hbm.at[0, :, pl.ds(0, _W)], sem_out
                ).wait()

        fire_pre(0)
        fire_in(0)

        def class_body(j, _):
            wait_pre()
            wait_in()

            @pl.when(j > 0)
            def _():
                wait_out()

            for c in range(nc):
                for k in range(_W // _L):
                    vbuf[c, 0, pl.ds(k * _L, _L)] = vpre[0, pl.ds(c * _W + k * _L, _L)]

            def shift_body(r, _):
                for c in range(nc):
                    copy_row(vbuf.at[c], 1 + n_ctx, vsuf.at[c], 0, r)
                return ()

            lax.fori_loop(0, n_suf, shift_body, (), unroll=False)

            fire_out(j)
            fire_pre(j + 1)
            fire_in(j + 1)
            return ()

        lax.fori_loop(0, cpw, class_body, (), unroll=False)

        wait_out()
        wait_pre()
        wait_in()

    return body(ctx, prefix, suffix)
